# baseline (speedup 1.0000x reference)
import functools

import numpy as np
import jax
import jax.numpy as jnp
from jax import lax
from jax.experimental import pallas as pl
from jax.experimental.pallas import tpu as pltpu
from jax.experimental.pallas import tpu_sc as plsc

B = 16384
D = 128
VOCABS = (4, 2, 2, 5, 3, 4, 4)
NF = len(VOCABS)
TOT = int(np.prod(VOCABS))
ROWS = sum(VOCABS)
RPAD = 24

STRIDES = tuple(int(np.prod(VOCABS[i + 1:])) for i in range(NF))
OFFSETS = tuple(int(sum(VOCABS[:i])) for i in range(NF))


def _build_onehot_t() -> np.ndarray:
    n = np.arange(TOT)
    u = np.zeros((RPAD, TOT), np.float32)
    for i in range(NF):
        c = (n // STRIDES[i]) % VOCABS[i]
        u[OFFSETS[i] + c, n] = 1.0
    return u


_UT = _build_onehot_t()


def _lut_body(u_ref, w_ref, t_ref):
    u = u_ref[...]
    w = w_ref[...]
    s = lax.dot_general(w, u, (((0,), (0,)), ((), ())),
                        preferred_element_type=jnp.float32,
                        precision=lax.Precision.HIGHEST)
    q = jnp.sum(w * w, axis=1, keepdims=True)
    t = jnp.sum(s * s, axis=0, keepdims=True)
    t = t - lax.dot_general(q, u, (((0,), (0,)), ((), ())),
                            preferred_element_type=jnp.float32,
                            precision=lax.Precision.HIGHEST)
    t_ref[...] = t


def _build_lut(wcat_t):
    return pl.pallas_call(
        _lut_body,
        out_shape=jax.ShapeDtypeStruct((1, TOT), jnp.float32),
    )(_UT, wcat_t)


_NC = 2
_NS = 16
_NW = _NC * _NS
BPW = B // _NW
_L = 16

@functools.cache
def _make_fm_gather():
    mesh = plsc.VectorSubcoreMesh(
        core_axis_name="c", subcore_axis_name="s", num_cores=_NC, num_subcores=_NS
    )

    @functools.partial(
        pl.kernel,
        out_type=jax.ShapeDtypeStruct((B,), jnp.float32),
        mesh=mesh,
        compiler_params=pltpu.CompilerParams(needs_layout_passes=False),
        scratch_types=[
            pltpu.VMEM((TOT,), jnp.float32),
            pltpu.VMEM((NF * BPW,), jnp.int32),
            pltpu.VMEM((BPW,), jnp.float32),
            pltpu.SemaphoreType.DMA,
            pltpu.SemaphoreType.DMA,
        ],
    )
    def _fm_gather(idx_hbm, lut_hbm, out_hbm, lut_v, idx_v, out_v, s_lut, s_idx):
        wid = lax.axis_index("s") * _NC + lax.axis_index("c")
        base = wid * BPW
        lut_cp = pltpu.make_async_copy(lut_hbm, lut_v, s_lut)
        idx_cps = [
            pltpu.make_async_copy(
                idx_hbm.at[pl.ds(i * B + base, BPW)],
                idx_v.at[pl.ds(i * BPW, BPW)],
                s_idx,
            )
            for i in range(NF)
        ]
        lut_cp.start()
        for c in idx_cps:
            c.start()
        for c in idx_cps:
            c.wait()
        flats = []
        for j in range(BPW // _L):
            f = idx_v[pl.ds(j * _L, _L)] * STRIDES[0]
            for i in range(1, NF):
                f = f + idx_v[pl.ds(i * BPW + j * _L, _L)] * STRIDES[i]
            flats.append(f)
        lut_cp.wait()
        for j, f in enumerate(flats):
            out_v[pl.ds(j * _L, _L)] = plsc.load_gather(lut_v, [f])
        pltpu.sync_copy(out_v, out_hbm.at[pl.ds(base, BPW)])

    return _fm_gather


def kernel(input, W1, W2, W3, W4, W5, W6, W7):
    idx = input.astype(jnp.int32).reshape(NF * B)
    wcat = jnp.concatenate([W1, W2, W3, W4, W5, W6, W7], axis=0)
    lut = _build_lut(wcat).reshape(TOT)
    out = _make_fm_gather()(idx, lut)
    return out.reshape(B, 1)


# speedup vs baseline: 1.0189x; 1.0189x over previous
import functools

import numpy as np
import jax
import jax.numpy as jnp
from jax import lax
from jax.experimental import pallas as pl
from jax.experimental.pallas import tpu as pltpu
from jax.experimental.pallas import tpu_sc as plsc

B = 16384
D = 128
VOCABS = (4, 2, 2, 5, 3, 4, 4)
NF = len(VOCABS)
TOT = int(np.prod(VOCABS))
ROWS = sum(VOCABS)
RPAD = 24

STRIDES = tuple(int(np.prod(VOCABS[i + 1:])) for i in range(NF))
OFFSETS = tuple(int(sum(VOCABS[:i])) for i in range(NF))


def _build_onehot_t() -> np.ndarray:
    n = np.arange(TOT)
    u = np.zeros((RPAD, TOT), np.float32)
    for i in range(NF):
        c = (n // STRIDES[i]) % VOCABS[i]
        u[OFFSETS[i] + c, n] = 1.0
    return u


_UT = _build_onehot_t()


def _lut_body(u_ref, w_ref, t_ref):
    u = u_ref[...]
    w = w_ref[...]
    s = lax.dot_general(w, u, (((0,), (0,)), ((), ())),
                        preferred_element_type=jnp.float32,
                        precision=lax.Precision.HIGHEST)
    q = jnp.sum(w * w, axis=1, keepdims=True)
    t = jnp.sum(s * s, axis=0, keepdims=True)
    t = t - lax.dot_general(q, u, (((0,), (0,)), ((), ())),
                            preferred_element_type=jnp.float32,
                            precision=lax.Precision.HIGHEST)
    t_ref[...] = t


def _build_lut(wcat_t):
    return pl.pallas_call(
        _lut_body,
        out_shape=jax.ShapeDtypeStruct((1, TOT), jnp.float32),
    )(_UT, wcat_t)


_NC = 2
_NS = 16
_NW = _NC * _NS
BPW = B // _NW
_L = 16

@functools.cache
def _make_fm_gather():
    mesh = plsc.VectorSubcoreMesh(
        core_axis_name="c", subcore_axis_name="s", num_cores=_NC, num_subcores=_NS
    )

    @functools.partial(
        pl.kernel,
        out_type=jax.ShapeDtypeStruct((B,), jnp.float32),
        mesh=mesh,
        compiler_params=pltpu.CompilerParams(needs_layout_passes=False),
        scratch_types=[
            pltpu.VMEM((TOT,), jnp.float32),
            pltpu.VMEM((NF * BPW,), jnp.int32),
            pltpu.VMEM((BPW,), jnp.float32),
            pltpu.SemaphoreType.DMA,
            pltpu.SemaphoreType.DMA,
        ],
    )
    def _fm_gather(idx_hbm, lut_hbm, out_hbm, lut_v, idx_v, out_v, s_lut, s_idx):
        wid = lax.axis_index("s") * _NC + lax.axis_index("c")
        base = wid * BPW
        lut_cp = pltpu.make_async_copy(lut_hbm, lut_v, s_lut)
        idx_cps = [
            pltpu.make_async_copy(
                idx_hbm.at[pl.ds(i * B + base, BPW)],
                idx_v.at[pl.ds(i * BPW, BPW)],
                s_idx,
            )
            for i in range(NF)
        ]
        lut_cp.start()
        for c in idx_cps:
            c.start()
        for c in idx_cps:
            c.wait()
        lut_cp.wait()
        for j in range(BPW // _L):
            f = idx_v[pl.ds(j * _L, _L)] * STRIDES[0]
            for i in range(1, NF):
                f = f + idx_v[pl.ds(i * BPW + j * _L, _L)] * STRIDES[i]
            out_v[pl.ds(j * _L, _L)] = plsc.load_gather(lut_v, [f])
        pltpu.sync_copy(out_v, out_hbm.at[pl.ds(base, BPW)])

    return _fm_gather


def kernel(input, W1, W2, W3, W4, W5, W6, W7):
    idx = input.astype(jnp.int32).reshape(NF * B)
    wcat = jnp.concatenate([W1, W2, W3, W4, W5, W6, W7], axis=0)
    lut = _build_lut(wcat).reshape(TOT)
    out = _make_fm_gather()(idx, lut)
    return out.reshape(B, 1)


# speedup vs baseline: 1.0581x; 1.0384x over previous
import functools

import numpy as np
import jax
import jax.numpy as jnp
from jax import lax
from jax.experimental import pallas as pl
from jax.experimental.pallas import tpu as pltpu
from jax.experimental.pallas import tpu_sc as plsc

B = 16384
D = 128
VOCABS = (4, 2, 2, 5, 3, 4, 4)
NF = len(VOCABS)
TOT = int(np.prod(VOCABS))
ROWS = sum(VOCABS)
RPAD = 32

STRIDES = tuple(int(np.prod(VOCABS[i + 1:])) for i in range(NF))
OFFSETS = tuple(int(sum(VOCABS[:i])) for i in range(NF))


def _build_onehot_t() -> np.ndarray:
    n = np.arange(TOT)
    u = np.zeros((RPAD, TOT), np.float32)
    for i in range(NF):
        c = (n // STRIDES[i]) % VOCABS[i]
        u[OFFSETS[i] + c, n] = 1.0
    return u


_UT = _build_onehot_t()


def _lut_body(u_ref, w_ref, t_ref):
    u = u_ref[...]
    w = w_ref[...]
    s = lax.dot_general(w, u, (((0,), (0,)), ((), ())),
                        preferred_element_type=jnp.float32,
                        precision=lax.Precision.HIGHEST)
    q = jnp.sum(w * w, axis=1, keepdims=True)
    t = jnp.sum(s * s, axis=0, keepdims=True)
    t = t - lax.dot_general(q, u, (((0,), (0,)), ((), ())),
                            preferred_element_type=jnp.float32,
                            precision=lax.Precision.HIGHEST)
    t_ref[...] = t


def _build_lut(wcat_t):
    return pl.pallas_call(
        _lut_body,
        out_shape=jax.ShapeDtypeStruct((1, TOT), jnp.float32),
    )(_UT, wcat_t)


_NC = 2
_NS = 16
_NW = _NC * _NS
BPW = B // _NW
_L = 16

@functools.cache
def _make_fm_gather():
    mesh = plsc.VectorSubcoreMesh(
        core_axis_name="c", subcore_axis_name="s", num_cores=_NC, num_subcores=_NS
    )

    @functools.partial(
        pl.kernel,
        out_type=jax.ShapeDtypeStruct((B,), jnp.float32),
        mesh=mesh,
        compiler_params=pltpu.CompilerParams(needs_layout_passes=False),
        scratch_types=[
            pltpu.VMEM((TOT,), jnp.float32),
            pltpu.VMEM((NF * BPW,), jnp.int32),
            pltpu.VMEM((BPW,), jnp.float32),
            pltpu.SemaphoreType.DMA,
            pltpu.SemaphoreType.DMA,
        ],
    )
    def _fm_gather(idx_hbm, lut_hbm, out_hbm, lut_v, idx_v, out_v, s_lut, s_idx):
        wid = lax.axis_index("s") * _NC + lax.axis_index("c")
        base = wid * BPW
        lut_cp = pltpu.make_async_copy(lut_hbm, lut_v, s_lut)
        idx_cps = [
            pltpu.make_async_copy(
                idx_hbm.at[pl.ds(i * B + base, BPW)],
                idx_v.at[pl.ds(i * BPW, BPW)],
                s_idx,
            )
            for i in range(NF)
        ]
        lut_cp.start()
        for c in idx_cps:
            c.start()
        for c in idx_cps:
            c.wait()
        lut_cp.wait()
        for j in range(BPW // _L):
            f = idx_v[pl.ds(j * _L, _L)] * STRIDES[0]
            for i in range(1, NF):
                f = f + idx_v[pl.ds(i * BPW + j * _L, _L)] * STRIDES[i]
            out_v[pl.ds(j * _L, _L)] = plsc.load_gather(lut_v, [f])
        pltpu.sync_copy(out_v, out_hbm.at[pl.ds(base, BPW)])

    return _fm_gather


def kernel(input, W1, W2, W3, W4, W5, W6, W7):
    idx = input.astype(jnp.int32).reshape(NF * B)
    wcat = jnp.concatenate([W1, W2, W3, W4, W5, W6, W7], axis=0)
    wcat = jnp.pad(wcat, ((0, RPAD - ROWS), (0, 0)))
    lut = _build_lut(wcat).reshape(TOT)
    out = _make_fm_gather()(idx, lut)
    return out.reshape(B, 1)
